# baseline (speedup 1.0000x reference)
import functools
import jax
import jax.numpy as jnp
from jax import lax
from jax.experimental import pallas as pl
from jax.experimental.pallas import tpu as pltpu
from jax.experimental.pallas import tpu_sc as plsc

N_NODES = 10000
N_EDGES = 320000
F1 = 128
F2 = 48
NCLASS = 40
NHID = 256

NC = 2
NS = 16
NP = 10240
K_EDGE = 128
EW = 10240
EP = NC * NS * EW
T_BLK = EW // K_EDGE
T_CH = 16
NCH = T_BLK // T_CH
RPS = NP // NS


@functools.lru_cache(maxsize=None)
def _make_spmm(feat, tc_tiling=True):
    mesh = plsc.VectorSubcoreMesh(core_axis_name="c", subcore_axis_name="s",
                                  num_cores=NC, num_subcores=NS)

    @functools.partial(
        pl.kernel,
        out_type=jax.ShapeDtypeStruct((NC * NP, feat), jnp.float32),
        mesh=mesh,
        compiler_params=pltpu.CompilerParams(use_tc_tiling_on_sc=tc_tiling),
        scratch_types=[
            pltpu.VMEM_SHARED((NP, feat), jnp.float32),
            pltpu.VMEM((T_CH, K_EDGE), jnp.int32),
            pltpu.VMEM((T_CH, K_EDGE), jnp.int32),
            pltpu.VMEM((T_CH * K_EDGE,), jnp.float32),
            pltpu.VMEM((K_EDGE, feat), jnp.float32),
            pltpu.VMEM((K_EDGE, feat), jnp.float32),
            pltpu.SemaphoreType.DMA,
            pltpu.SemaphoreType.DMA,
            pltpu.SemaphoreType.DMA,
            pltpu.SemaphoreType.DMA,
        ],
    )
    def spmm(x_hbm, row2_hbm, col2_hbm, w_hbm, z_hbm, out_hbm,
             acc, cslab, rslab, wslab, buf0, buf1, gs0, gs1, ss0, ss1):
        c = lax.axis_index("c")
        s = lax.axis_index("s")
        wid = c * NS + s

        pltpu.sync_copy(z_hbm, acc.at[pl.ds(s * RPS, RPS)])
        plsc.subcore_barrier()

        def scale(buf, blk):
            def grp(g, _):
                wv = wslab[pl.ds(blk * K_EDGE + g * 16, 16)]
                for e in range(16):
                    j = g * 16 + e
                    wj = wv[e]
                    for t in range(feat // 16):
                        sl = pl.ds(t * 16, 16)
                        buf[j, sl] = buf[j, sl] * wj
                return 0

            lax.fori_loop(0, K_EDGE // 16, grp, 0)

        def chunk(ch, _):
            cb = wid * T_BLK + ch * T_CH
            pltpu.sync_copy(col2_hbm.at[pl.ds(cb, T_CH)], cslab)
            pltpu.sync_copy(row2_hbm.at[pl.ds(cb, T_CH)], rslab)
            pltpu.sync_copy(w_hbm.at[pl.ds(wid * EW + ch * T_CH * K_EDGE,
                                           T_CH * K_EDGE)], wslab)

            def gather_start(j, buf, sem):
                for h in range(2):
                    hs = pl.ds(h * (K_EDGE // 2), K_EDGE // 2)
                    pltpu.async_copy(x_hbm.at[cslab.at[j, hs]],
                                     buf.at[hs], sem)

            def gather_wait(j, buf, sem):
                for h in range(2):
                    hs = pl.ds(h * (K_EDGE // 2), K_EDGE // 2)
                    pltpu.make_async_copy(x_hbm.at[cslab.at[j, hs]],
                                          buf.at[hs], sem).wait()

            gather_start(0, buf0, gs0)

            def pair(p, _):
                a = 2 * p
                b = a + 1
                gather_wait(a, buf0, gs0)

                @pl.when(p > 0)
                def _():
                    pltpu.make_async_copy(
                        buf1, acc.at[rslab.at[0]], ss1).wait()

                gather_start(b, buf1, gs1)
                scale(buf0, a)
                pltpu.async_copy(buf0, acc.at[rslab.at[a]], ss0, add=True)

                gather_wait(b, buf1, gs1)

                @pl.when(p < T_CH // 2 - 1)
                def _():
                    pltpu.make_async_copy(
                        buf0, acc.at[rslab.at[0]], ss0).wait()
                    gather_start(a + 2, buf0, gs0)

                scale(buf1, b)
                pltpu.async_copy(buf1, acc.at[rslab.at[b]], ss1, add=True)
                return 0

            lax.fori_loop(0, T_CH // 2, pair, 0)
            pltpu.make_async_copy(buf0, acc.at[rslab.at[0]], ss0).wait()
            pltpu.make_async_copy(buf1, acc.at[rslab.at[0]], ss1).wait()
            return 0

        lax.fori_loop(0, NCH, chunk, 0)
        plsc.subcore_barrier()

        pltpu.sync_copy(acc.at[pl.ds(s * RPS, RPS)],
                        out_hbm.at[pl.ds(c * NP + s * RPS, RPS)])

    return spmm


_BM = 1000


def _mm_body(p0_ref, p1_ref, w1_ref, b1_ref, w2_ref, out_ref):
    agg = p0_ref[0] + p1_ref[0]
    h = jnp.dot(agg, w1_ref[...], preferred_element_type=jnp.float32)
    h = jnp.maximum(h + b1_ref[...], 0.0)
    out_ref[...] = jnp.dot(h, w2_ref[...], preferred_element_type=jnp.float32)


def _ls_body(q0_ref, q1_ref, b2_ref, out_ref):
    z = q0_ref[0] + q1_ref[0] + b2_ref[...]
    m = jnp.max(z, axis=1, keepdims=True)
    lse = jnp.log(jnp.sum(jnp.exp(z - m), axis=1, keepdims=True)) + m
    out_ref[...] = z[:, :NCLASS] - lse


def kernel(x, edge_index, edge_weight, W1, b1, W2, b2):
    row = edge_index[0]
    col = edge_index[1]
    pad = EP - N_EDGES
    parange = jnp.arange(pad, dtype=jnp.int32)
    rowp = jnp.concatenate([row, N_NODES + parange % (NP - N_NODES)])
    colp = jnp.concatenate([col, parange % N_NODES])
    rowp = rowp.reshape(EP // K_EDGE, K_EDGE)
    colp = colp.reshape(EP // K_EDGE, K_EDGE)
    wp = jnp.pad(edge_weight, (0, pad))

    part1 = _make_spmm(F1)(x, rowp, colp, wp,
                           jnp.zeros((RPS, F1), jnp.float32))
    part1 = part1.reshape(NC, NP, F1)

    W2p = jnp.pad(W2, ((0, 0), (0, F2 - NCLASS)))
    nblk = N_NODES // _BM
    s2 = pl.pallas_call(
        _mm_body,
        grid=(nblk,),
        in_specs=[
            pl.BlockSpec((1, _BM, F1), lambda i: (0, i, 0)),
            pl.BlockSpec((1, _BM, F1), lambda i: (1, i, 0)),
            pl.BlockSpec((F1, NHID), lambda i: (0, 0)),
            pl.BlockSpec((1, NHID), lambda i: (0, 0)),
            pl.BlockSpec((NHID, F2), lambda i: (0, 0)),
        ],
        out_specs=pl.BlockSpec((_BM, F2), lambda i: (i, 0)),
        out_shape=jax.ShapeDtypeStruct((N_NODES, F2), jnp.float32),
    )(part1, part1, W1, b1[None, :], W2p)

    part2 = _make_spmm(F2, tc_tiling=False)(s2, rowp, colp, wp,
                           jnp.zeros((RPS, F2), jnp.float32))
    part2 = part2.reshape(NC, NP, F2)

    b2p = jnp.full((F2,), -1e30, jnp.float32).at[:NCLASS].set(b2)
    out = pl.pallas_call(
        _ls_body,
        grid=(nblk,),
        in_specs=[
            pl.BlockSpec((1, _BM, F2), lambda i: (0, i, 0)),
            pl.BlockSpec((1, _BM, F2), lambda i: (1, i, 0)),
            pl.BlockSpec((1, F2), lambda i: (0, 0)),
        ],
        out_specs=pl.BlockSpec((_BM, NCLASS), lambda i: (i, 0)),
        out_shape=jax.ShapeDtypeStruct((N_NODES, NCLASS), jnp.float32),
    )(part2, part2, b2p[None, :])

    return out


# speedup vs baseline: 1.0133x; 1.0133x over previous
import functools
import jax
import jax.numpy as jnp
from jax import lax
from jax.experimental import pallas as pl
from jax.experimental.pallas import tpu as pltpu
from jax.experimental.pallas import tpu_sc as plsc

N_NODES = 10000
N_EDGES = 320000
F1 = 128
F2 = 48
NCLASS = 40
NHID = 256

NC = 2
NS = 16
NP = 10240
K_EDGE = 128
EW = 10240
EP = NC * NS * EW
T_BLK = EW // K_EDGE
T_CH = 16
NCH = T_BLK // T_CH
RPS = NP // NS


@functools.lru_cache(maxsize=None)
def _make_spmm(feat, tc_tiling=True):
    mesh = plsc.VectorSubcoreMesh(core_axis_name="c", subcore_axis_name="s",
                                  num_cores=NC, num_subcores=NS)

    @functools.partial(
        pl.kernel,
        out_type=jax.ShapeDtypeStruct((NC * NP, feat), jnp.float32),
        mesh=mesh,
        compiler_params=pltpu.CompilerParams(use_tc_tiling_on_sc=tc_tiling),
        scratch_types=[
            pltpu.VMEM_SHARED((NP, feat), jnp.float32),
            pltpu.VMEM((T_CH, K_EDGE), jnp.int32),
            pltpu.VMEM((T_CH, K_EDGE), jnp.int32),
            pltpu.VMEM((T_CH * K_EDGE,), jnp.float32),
            pltpu.VMEM((K_EDGE, feat), jnp.float32),
            pltpu.VMEM((K_EDGE, feat), jnp.float32),
            pltpu.SemaphoreType.DMA,
            pltpu.SemaphoreType.DMA,
            pltpu.SemaphoreType.DMA,
            pltpu.SemaphoreType.DMA,
        ],
    )
    def spmm(x_hbm, row2_hbm, col2_hbm, w_hbm, z_hbm, out_hbm,
             acc, cslab, rslab, wslab, buf0, buf1, gs0, gs1, ss0, ss1):
        c = lax.axis_index("c")
        s = lax.axis_index("s")
        wid = c * NS + s

        pltpu.sync_copy(z_hbm, acc.at[pl.ds(s * RPS, RPS)])
        plsc.subcore_barrier()

        def scale(buf, blk):
            def grp(g, _):
                wv = wslab[pl.ds(blk * K_EDGE + g * 16, 16)]
                for e in range(16):
                    j = g * 16 + e
                    wj = wv[e]
                    for t in range(feat // 16):
                        sl = pl.ds(t * 16, 16)
                        buf[j, sl] = buf[j, sl] * wj
                return 0

            lax.fori_loop(0, K_EDGE // 16, grp, 0)

        def chunk(ch, _):
            cb = wid * T_BLK + ch * T_CH
            pltpu.sync_copy(col2_hbm.at[pl.ds(cb, T_CH)], cslab)
            pltpu.sync_copy(row2_hbm.at[pl.ds(cb, T_CH)], rslab)
            pltpu.sync_copy(w_hbm.at[pl.ds(wid * EW + ch * T_CH * K_EDGE,
                                           T_CH * K_EDGE)], wslab)

            def gather_start(j, buf, sem):
                for h in range(2):
                    hs = pl.ds(h * (K_EDGE // 2), K_EDGE // 2)
                    pltpu.async_copy(x_hbm.at[cslab.at[j, hs]],
                                     buf.at[hs], sem)

            def gather_wait(j, buf, sem):
                for h in range(2):
                    hs = pl.ds(h * (K_EDGE // 2), K_EDGE // 2)
                    pltpu.make_async_copy(x_hbm.at[cslab.at[j, hs]],
                                          buf.at[hs], sem).wait()

            gather_start(0, buf0, gs0)

            def pair(p, _):
                a = 2 * p
                b = a + 1
                gather_wait(a, buf0, gs0)

                @pl.when(p > 0)
                def _():
                    pltpu.make_async_copy(
                        buf1, acc.at[rslab.at[0]], ss1).wait()

                gather_start(b, buf1, gs1)
                scale(buf0, a)
                pltpu.async_copy(buf0, acc.at[rslab.at[a]], ss0, add=True)

                gather_wait(b, buf1, gs1)

                @pl.when(p < T_CH // 2 - 1)
                def _():
                    pltpu.make_async_copy(
                        buf0, acc.at[rslab.at[0]], ss0).wait()
                    gather_start(a + 2, buf0, gs0)

                scale(buf1, b)
                pltpu.async_copy(buf1, acc.at[rslab.at[b]], ss1, add=True)
                return 0

            lax.fori_loop(0, T_CH // 2, pair, 0)
            pltpu.make_async_copy(buf0, acc.at[rslab.at[0]], ss0).wait()
            pltpu.make_async_copy(buf1, acc.at[rslab.at[0]], ss1).wait()
            return 0

        lax.fori_loop(0, NCH, chunk, 0)
        plsc.subcore_barrier()

        pltpu.sync_copy(acc.at[pl.ds(s * RPS, RPS)],
                        out_hbm.at[pl.ds(c * NP + s * RPS, RPS)])

    return spmm


_BM = 2000


def _mm_body(p0_ref, p1_ref, w1_ref, b1_ref, w2_ref, out_ref):
    agg = p0_ref[0] + p1_ref[0]
    h = jnp.dot(agg, w1_ref[...], preferred_element_type=jnp.float32)
    h = jnp.maximum(h + b1_ref[...], 0.0)
    out_ref[...] = jnp.dot(h, w2_ref[...], preferred_element_type=jnp.float32)


def _ls_body(q0_ref, q1_ref, b2_ref, out_ref):
    z = q0_ref[0] + q1_ref[0] + b2_ref[...]
    m = jnp.max(z, axis=1, keepdims=True)
    lse = jnp.log(jnp.sum(jnp.exp(z - m), axis=1, keepdims=True)) + m
    out_ref[...] = z[:, :NCLASS] - lse


def kernel(x, edge_index, edge_weight, W1, b1, W2, b2):
    row = edge_index[0]
    col = edge_index[1]
    pad = EP - N_EDGES
    parange = jnp.arange(pad, dtype=jnp.int32)
    rowp = jnp.concatenate([row, N_NODES + parange % (NP - N_NODES)])
    colp = jnp.concatenate([col, parange % N_NODES])
    rowp = rowp.reshape(EP // K_EDGE, K_EDGE)
    colp = colp.reshape(EP // K_EDGE, K_EDGE)
    wp = jnp.pad(edge_weight, (0, pad))

    part1 = _make_spmm(F1)(x, rowp, colp, wp,
                           jnp.zeros((RPS, F1), jnp.float32))
    part1 = part1.reshape(NC, NP, F1)

    W2p = jnp.pad(W2, ((0, 0), (0, F2 - NCLASS)))
    nblk = N_NODES // _BM
    s2 = pl.pallas_call(
        _mm_body,
        grid=(nblk,),
        in_specs=[
            pl.BlockSpec((1, _BM, F1), lambda i: (0, i, 0)),
            pl.BlockSpec((1, _BM, F1), lambda i: (1, i, 0)),
            pl.BlockSpec((F1, NHID), lambda i: (0, 0)),
            pl.BlockSpec((1, NHID), lambda i: (0, 0)),
            pl.BlockSpec((NHID, F2), lambda i: (0, 0)),
        ],
        out_specs=pl.BlockSpec((_BM, F2), lambda i: (i, 0)),
        out_shape=jax.ShapeDtypeStruct((N_NODES, F2), jnp.float32),
    )(part1, part1, W1, b1[None, :], W2p)

    part2 = _make_spmm(F2, tc_tiling=False)(s2, rowp, colp, wp,
                           jnp.zeros((RPS, F2), jnp.float32))
    part2 = part2.reshape(NC, NP, F2)

    b2p = jnp.full((F2,), -1e30, jnp.float32).at[:NCLASS].set(b2)
    out = pl.pallas_call(
        _ls_body,
        grid=(nblk,),
        in_specs=[
            pl.BlockSpec((1, _BM, F2), lambda i: (0, i, 0)),
            pl.BlockSpec((1, _BM, F2), lambda i: (1, i, 0)),
            pl.BlockSpec((1, F2), lambda i: (0, 0)),
        ],
        out_specs=pl.BlockSpec((_BM, NCLASS), lambda i: (i, 0)),
        out_shape=jax.ShapeDtypeStruct((N_NODES, NCLASS), jnp.float32),
    )(part2, part2, b2p[None, :])

    return out
